# Initial kernel scaffold; baseline (speedup 1.0000x reference)
import jax
import jax.numpy as jnp
from jax.experimental import pallas as pl


def kernel(x, edge_index, W1, b1, W2, b2):
    raise NotImplementedError("write your pallas kernel here")


# speedup vs baseline: 108.5331x; 108.5331x over previous
import functools

import jax
import jax.numpy as jnp
from jax import lax
from jax.experimental import pallas as pl
from jax.experimental.pallas import tpu as pltpu
from jax.experimental.pallas import tpu_sc as plsc

N_NODES = 100000
N_EDGES = 6400000

LANES = 128
NP = 100352
NPR = NP // LANES
DUMMY = N_NODES

NW = 32
K_ROWS = 8
R_ROWS = 1568
N_GROUPS = R_ROWS // K_ROWS
ROWS_TOTAL = NW * R_ROWS
E_PAD = ROWS_TOTAL * 128


@functools.lru_cache(maxsize=None)
def _sc_edge_pass(with_table):
    mesh = plsc.VectorSubcoreMesh(core_axis_name="c", subcore_axis_name="s")
    scratch = []
    if with_table:
        scratch.append(pltpu.VMEM((NP,), jnp.float32))
        scratch.append(pltpu.VMEM((K_ROWS, 128), jnp.int32))
    scratch += [
        pltpu.VMEM((K_ROWS, 128), jnp.int32),
        pltpu.VMEM((K_ROWS, 128), jnp.float32),
        pltpu.VMEM_SHARED((NP,), jnp.float32),
        pltpu.SemaphoreType.DMA,
    ]

    def body(*refs):
        if with_table:
            (tab_hbm, src_hbm, dst_hbm, zeros_hbm, out_hbm,
             ytab, sbuf, dbuf, vbuf, acc, ssem) = refs
        else:
            (dst_hbm, zeros_hbm, out_hbm, dbuf, vbuf, acc, ssem) = refs
        c = lax.axis_index("c")
        s = lax.axis_index("s")
        wid = c * 16 + s

        @pl.when(s == 0)
        def _():
            pltpu.sync_copy(zeros_hbm, acc)

        if with_table:
            pltpu.sync_copy(tab_hbm, ytab)
        else:
            ones = jnp.full((16,), 1.0, jnp.float32)
            for i in range(8):
                vbuf[0, pl.ds(i * 16, 16)] = ones
        plsc.subcore_barrier()

        base_row = wid * R_ROWS

        def group(g, carry):
            row0 = base_row + g * K_ROWS
            pltpu.sync_copy(dst_hbm.at[pl.ds(row0, K_ROWS)], dbuf)
            if with_table:
                pltpu.sync_copy(src_hbm.at[pl.ds(row0, K_ROWS)], sbuf)
                for j in range(K_ROWS):
                    for i in range(8):
                        idx = sbuf[j, pl.ds(i * 16, 16)]
                        vbuf[j, pl.ds(i * 16, 16)] = plsc.load_gather(
                            ytab, [idx])
            cps = []
            for j in range(K_ROWS):
                vrow = vbuf.at[j] if with_table else vbuf.at[0]
                cps.append(pltpu.async_copy(
                    vrow, acc.at[dbuf.at[j]], ssem, add=True))
            for cp in cps:
                cp.wait()
            return carry

        lax.fori_loop(0, N_GROUPS, group, 0)
        plsc.subcore_barrier()

        @pl.when(s == 0)
        def _():
            pltpu.sync_copy(acc, out_hbm.at[c])

    return pl.kernel(
        body,
        out_type=jax.ShapeDtypeStruct((2, NP), jnp.float32),
        mesh=mesh,
        scratch_types=scratch,
        compiler_params=pltpu.CompilerParams(needs_layout_passes=False),
    )


_GRID = (NPR // 8,)
_BLK = pl.BlockSpec((8, 128), lambda g: (g, 0))
_SMEM = pl.BlockSpec(memory_space=pltpu.SMEM)


def _tc_a_body(s0, s1, x, dinv_o, y_o):
    deg = s0[...] + s1[...] + 1.0
    dinv = lax.rsqrt(deg)
    dinv_o[...] = dinv
    y_o[...] = x[...] * dinv


@jax.jit
def _tc_a(s0, s1, x):
    return pl.pallas_call(
        _tc_a_body,
        grid=_GRID,
        in_specs=[_BLK, _BLK, _BLK],
        out_specs=[_BLK, _BLK],
        out_shape=[jax.ShapeDtypeStruct((NPR, 128), jnp.float32)] * 2,
    )(s0, s1, x)


def _tc_b_body(s0, s1, dinv_r, x, w1, b1, w2, z0_o, z1_o):
    dinv = dinv_r[...]
    t = dinv * (s0[...] + s1[...]) + x[...] * dinv * dinv
    a0 = jnp.zeros_like(t)
    a1 = jnp.zeros_like(t)
    for j in range(16):
        h = jnp.maximum(t * w1[0, j] + b1[j], 0.0)
        a0 = a0 + h * w2[j, 0]
        a1 = a1 + h * w2[j, 1]
    z0_o[...] = a0 * dinv
    z1_o[...] = a1 * dinv


@jax.jit
def _tc_b(s0, s1, dinv, x, W1, b1, W2):
    return pl.pallas_call(
        _tc_b_body,
        grid=_GRID,
        in_specs=[_BLK, _BLK, _BLK, _BLK, _SMEM, _SMEM, _SMEM],
        out_specs=[_BLK, _BLK],
        out_shape=[jax.ShapeDtypeStruct((NPR, 128), jnp.float32)] * 2,
    )(s0, s1, dinv, x, W1, b1, W2)


def _tc_c_body(p00, p01, p10, p11, dinv_r, z0, z1, b2, o0, o1):
    dinv = dinv_r[...]
    a0 = dinv * (p00[...] + p01[...]) + z0[...] * dinv + b2[0]
    a1 = dinv * (p10[...] + p11[...]) + z1[...] * dinv + b2[1]
    m = jnp.maximum(a0, a1)
    lse = m + jnp.log(jnp.exp(a0 - m) + jnp.exp(a1 - m))
    o0[...] = a0 - lse
    o1[...] = a1 - lse


@jax.jit
def _tc_c(p00, p01, p10, p11, dinv, z0, z1, b2):
    return pl.pallas_call(
        _tc_c_body,
        grid=_GRID,
        in_specs=[_BLK] * 7 + [_SMEM],
        out_specs=[_BLK, _BLK],
        out_shape=[jax.ShapeDtypeStruct((NPR, 128), jnp.float32)] * 2,
    )(p00, p01, p10, p11, dinv, z0, z1, b2)


def kernel(x, edge_index, W1, b1, W2, b2):
    src = edge_index[0].astype(jnp.int32)
    dst = edge_index[1].astype(jnp.int32)
    pad = E_PAD - N_EDGES
    src2d = jnp.concatenate(
        [src, jnp.zeros((pad,), jnp.int32)]).reshape(ROWS_TOTAL, 128)
    dst2d = jnp.concatenate(
        [dst, jnp.full((pad,), DUMMY, jnp.int32)]).reshape(ROWS_TOTAL, 128)
    zeros = jnp.zeros((NP,), jnp.float32)
    xp = jnp.pad(x[:, 0], (0, NP - N_NODES)).reshape(NPR, 128)

    degp = _sc_edge_pass(False)(dst2d, zeros)
    dinv, y = _tc_a(degp[0].reshape(NPR, 128), degp[1].reshape(NPR, 128), xp)

    val_pass = _sc_edge_pass(True)
    sp = val_pass(y.reshape(NP), src2d, dst2d, zeros)
    z0, z1 = _tc_b(sp[0].reshape(NPR, 128), sp[1].reshape(NPR, 128),
                   dinv, xp, W1, b1, W2)

    p0 = val_pass(z0.reshape(NP), src2d, dst2d, zeros)
    p1 = val_pass(z1.reshape(NP), src2d, dst2d, zeros)
    o0, o1 = _tc_c(p0[0].reshape(NPR, 128), p0[1].reshape(NPR, 128),
                   p1[0].reshape(NPR, 128), p1[1].reshape(NPR, 128),
                   dinv, z0, z1, b2)
    return jnp.stack([o0.reshape(NP)[:N_NODES],
                      o1.reshape(NP)[:N_NODES]], axis=-1)


# speedup vs baseline: 264.5254x; 2.4373x over previous
import functools

import jax
import jax.numpy as jnp
from jax import lax
from jax.experimental import pallas as pl
from jax.experimental.pallas import tpu as pltpu
from jax.experimental.pallas import tpu_sc as plsc

N_NODES = 100000
N_EDGES = 6400000

LANES = 128
NP = 100352
NPR = NP // LANES

NW = 32
K_ROWS = 8
ROWS_TOTAL = N_EDGES // 128
N_GROUPS = ROWS_TOTAL // K_ROWS
SLOTS = 198


@functools.lru_cache(maxsize=None)
def _sc_edge_pass(with_table):
    mesh = plsc.VectorSubcoreMesh(core_axis_name="c", subcore_axis_name="s")
    scratch = []
    if with_table:
        scratch.append(pltpu.VMEM((NP,), jnp.float32))
        scratch += [pltpu.VMEM((K_ROWS, 128), jnp.int32)] * 3
    scratch += [pltpu.VMEM((K_ROWS, 128), jnp.int32)] * 3
    scratch += [pltpu.VMEM((K_ROWS, 128), jnp.float32)] * 3
    scratch += [pltpu.VMEM_SHARED((NP,), jnp.float32)]
    scratch += [pltpu.SemaphoreType.DMA] * 3
    scratch += [pltpu.SemaphoreType.DMA] * 3

    def body(*refs):
        if with_table:
            (tab_hbm, ei_hbm, zeros_hbm, out_hbm, ytab,
             sb0, sb1, sb2, db0, db1, db2, vb0, vb1, vb2,
             acc, l0, l1, l2, t0, t1, t2) = refs
            sbuf = (sb0, sb1, sb2)
        else:
            (ei_hbm, zeros_hbm, out_hbm,
             db0, db1, db2, vb0, vb1, vb2,
             acc, l0, l1, l2, t0, t1, t2) = refs
        dbuf = (db0, db1, db2)
        vbuf = (vb0, vb1, vb2)
        lsem = (l0, l1, l2)
        ssem = (t0, t1, t2)
        c = lax.axis_index("c")
        s = lax.axis_index("s")
        wid = c * 16 + s

        @pl.when(s == 0)
        def _():
            pltpu.sync_copy(zeros_hbm, acc)

        if with_table:
            pltpu.sync_copy(tab_hbm, ytab)
        else:
            ones = jnp.full((16,), 1.0, jnp.float32)
            for p in range(3):
                for j in range(K_ROWS):
                    for i in range(8):
                        vbuf[p][j, pl.ds(i * 16, 16)] = ones
        plsc.subcore_barrier()

        def issue_load(k, p):
            g = wid + 32 * k

            @pl.when(g < N_GROUPS)
            def _():
                row0 = g * K_ROWS
                pltpu.async_copy(
                    ei_hbm.at[1, pl.ds(row0, K_ROWS)], dbuf[p], lsem[p])
                if with_table:
                    pltpu.async_copy(
                        ei_hbm.at[0, pl.ds(row0, K_ROWS)], sbuf[p], lsem[p])

        def wait_load(p):
            pltpu.make_async_copy(
                ei_hbm.at[1, pl.ds(0, K_ROWS)], dbuf[p], lsem[p]).wait()
            if with_table:
                pltpu.make_async_copy(
                    ei_hbm.at[0, pl.ds(0, K_ROWS)], sbuf[p], lsem[p]).wait()

        def drain_scatters(p):
            for j in range(K_ROWS):
                pltpu.make_async_copy(
                    vbuf[p].at[0], acc.at[dbuf[p].at[0]], ssem[p]).wait()

        issue_load(0, 0)

        def step(t, carry):
            for j3 in range(3):
                k = 3 * t + j3
                g = wid + 32 * k

                @pl.when((k >= 2) & (wid + 32 * (k - 2) < N_GROUPS))
                def _():
                    drain_scatters((j3 + 1) % 3)

                issue_load(k + 1, (j3 + 1) % 3)

                @pl.when(g < N_GROUPS)
                def _():
                    wait_load(j3)
                    if with_table:
                        for j in range(K_ROWS):
                            for i in range(8):
                                idx = sbuf[j3][j, pl.ds(i * 16, 16)]
                                vbuf[j3][j, pl.ds(i * 16, 16)] = (
                                    plsc.load_gather(ytab, [idx]))
                    for j in range(K_ROWS):
                        pltpu.async_copy(
                            vbuf[j3].at[j], acc.at[dbuf[j3].at[j]],
                            ssem[j3], add=True)
            return carry

        lax.fori_loop(0, SLOTS // 3, step, 0)
        plsc.subcore_barrier()

        @pl.when(s == 0)
        def _():
            pltpu.sync_copy(acc, out_hbm.at[c])

    return pl.kernel(
        body,
        out_type=jax.ShapeDtypeStruct((2, NP), jnp.float32),
        mesh=mesh,
        scratch_types=scratch,
        compiler_params=pltpu.CompilerParams(needs_layout_passes=False),
    )


_GRID = (NPR // 8,)
_BLK = pl.BlockSpec((8, 128), lambda g: (g, 0))
_SMEM = pl.BlockSpec(memory_space=pltpu.SMEM)


def _tc_a_body(s0, s1, x, dinv_o, y_o):
    deg = s0[...] + s1[...] + 1.0
    dinv = lax.rsqrt(deg)
    dinv_o[...] = dinv
    y_o[...] = x[...] * dinv


@jax.jit
def _tc_a(s0, s1, x):
    return pl.pallas_call(
        _tc_a_body,
        grid=_GRID,
        in_specs=[_BLK, _BLK, _BLK],
        out_specs=[_BLK, _BLK],
        out_shape=[jax.ShapeDtypeStruct((NPR, 128), jnp.float32)] * 2,
    )(s0, s1, x)


def _tc_b_body(s0, s1, dinv_r, x, w1, b1, w2, z0_o, z1_o):
    dinv = dinv_r[...]
    t = dinv * (s0[...] + s1[...]) + x[...] * dinv * dinv
    a0 = jnp.zeros_like(t)
    a1 = jnp.zeros_like(t)
    for j in range(16):
        h = jnp.maximum(t * w1[0, j] + b1[j], 0.0)
        a0 = a0 + h * w2[j, 0]
        a1 = a1 + h * w2[j, 1]
    z0_o[...] = a0 * dinv
    z1_o[...] = a1 * dinv


@jax.jit
def _tc_b(s0, s1, dinv, x, W1, b1, W2):
    return pl.pallas_call(
        _tc_b_body,
        grid=_GRID,
        in_specs=[_BLK, _BLK, _BLK, _BLK, _SMEM, _SMEM, _SMEM],
        out_specs=[_BLK, _BLK],
        out_shape=[jax.ShapeDtypeStruct((NPR, 128), jnp.float32)] * 2,
    )(s0, s1, dinv, x, W1, b1, W2)


def _tc_c_body(p00, p01, p10, p11, dinv_r, z0, z1, b2, o0, o1):
    dinv = dinv_r[...]
    a0 = dinv * (p00[...] + p01[...]) + z0[...] * dinv + b2[0]
    a1 = dinv * (p10[...] + p11[...]) + z1[...] * dinv + b2[1]
    m = jnp.maximum(a0, a1)
    lse = m + jnp.log(jnp.exp(a0 - m) + jnp.exp(a1 - m))
    o0[...] = a0 - lse
    o1[...] = a1 - lse


@jax.jit
def _tc_c(p00, p01, p10, p11, dinv, z0, z1, b2):
    return pl.pallas_call(
        _tc_c_body,
        grid=_GRID,
        in_specs=[_BLK] * 7 + [_SMEM],
        out_specs=[_BLK, _BLK],
        out_shape=[jax.ShapeDtypeStruct((NPR, 128), jnp.float32)] * 2,
    )(p00, p01, p10, p11, dinv, z0, z1, b2)


def kernel(x, edge_index, W1, b1, W2, b2):
    ei3 = edge_index.astype(jnp.int32).reshape(2, ROWS_TOTAL, 128)
    zeros = jnp.zeros((NP,), jnp.float32)
    xp = jnp.pad(x[:, 0], (0, NP - N_NODES)).reshape(NPR, 128)

    degp = _sc_edge_pass(False)(ei3, zeros)
    dinv, y = _tc_a(degp[0].reshape(NPR, 128), degp[1].reshape(NPR, 128), xp)

    val_pass = _sc_edge_pass(True)
    sp = val_pass(y.reshape(NP), ei3, zeros)
    z0, z1 = _tc_b(sp[0].reshape(NPR, 128), sp[1].reshape(NPR, 128),
                   dinv, xp, W1, b1, W2)

    p0 = val_pass(z0.reshape(NP), ei3, zeros)
    p1 = val_pass(z1.reshape(NP), ei3, zeros)
    o0, o1 = _tc_c(p0[0].reshape(NPR, 128), p0[1].reshape(NPR, 128),
                   p1[0].reshape(NPR, 128), p1[1].reshape(NPR, 128),
                   dinv, z0, z1, b2)
    return jnp.stack([o0.reshape(NP)[:N_NODES],
                      o1.reshape(NP)[:N_NODES]], axis=-1)


# speedup vs baseline: 359.7612x; 1.3600x over previous
import functools

import jax
import jax.numpy as jnp
from jax import lax
from jax.experimental import pallas as pl
from jax.experimental.pallas import tpu as pltpu
from jax.experimental.pallas import tpu_sc as plsc

N_NODES = 100000
N_EDGES = 6400000

LANES = 128
NP = 100352
NPR = NP // LANES

NW = 32
ROWS_TOTAL = N_EDGES // 128


@functools.lru_cache(maxsize=None)
def _sc_edge_pass(mode):
    with_table = mode != "deg"
    nacc = 2 if mode == "z" else 1
    kr = 8 if mode == "z" else 16
    n_groups = ROWS_TOTAL // kr
    slots = -(-(-(-n_groups // 32)) // 3) * 3
    mesh = plsc.VectorSubcoreMesh(core_axis_name="c", subcore_axis_name="s")
    tab_dtype = jnp.int32 if mode == "z" else jnp.float32
    scratch = []
    if with_table:
        scratch.append(pltpu.VMEM((NP,), tab_dtype))
        scratch += [pltpu.VMEM((kr * 128,), jnp.int32)] * 3
    scratch += [pltpu.VMEM((kr * 128,), jnp.int32)] * 3
    scratch += [pltpu.VMEM((kr * 128,), jnp.float32)] * (3 * nacc)
    scratch += [pltpu.VMEM_SHARED((NP,), jnp.float32)] * nacc
    scratch += [pltpu.SemaphoreType.DMA] * 3
    scratch += [pltpu.SemaphoreType.DMA] * 3

    def body(*refs):
        it = iter(refs)
        if with_table:
            tab_hbm = next(it)
        ei_hbm = next(it)
        zeros_hbm = next(it)
        out_hbm = next(it)
        if with_table:
            tab = next(it)
            sbuf = (next(it), next(it), next(it))
        dbuf = (next(it), next(it), next(it))
        vbuf = tuple(tuple(next(it) for _ in range(3)) for _ in range(nacc))
        acc = tuple(next(it) for _ in range(nacc))
        lsem = (next(it), next(it), next(it))
        ssem = (next(it), next(it), next(it))
        c = lax.axis_index("c")
        s = lax.axis_index("s")
        wid = c * 16 + s

        @pl.when(s == 0)
        def _():
            for a in acc:
                pltpu.sync_copy(zeros_hbm, a)

        if with_table:
            pltpu.sync_copy(tab_hbm, tab)
        else:
            ones = jnp.full((16,), 1.0, jnp.float32)
            for p in range(3):
                for i in range(kr * 8):
                    vbuf[0][p][pl.ds(i * 16, 16)] = ones
        plsc.subcore_barrier()

        def issue_load(k, p):
            g = wid + 32 * k

            @pl.when(g < n_groups)
            def _():
                e0 = g * (kr * 128)
                pltpu.async_copy(
                    ei_hbm.at[1, pl.ds(e0, kr * 128)], dbuf[p], lsem[p])
                if with_table:
                    pltpu.async_copy(
                        ei_hbm.at[0, pl.ds(e0, kr * 128)], sbuf[p], lsem[p])

        def wait_load(p):
            pltpu.make_async_copy(
                ei_hbm.at[1, pl.ds(0, kr * 128)], dbuf[p], lsem[p]).wait()
            if with_table:
                pltpu.make_async_copy(
                    ei_hbm.at[0, pl.ds(0, kr * 128)], sbuf[p], lsem[p]).wait()

        def drain_scatters(p):
            for a in range(nacc):
                for j in range(kr):
                    pltpu.make_async_copy(
                        vbuf[a][p].at[pl.ds(0, 128)],
                        acc[a].at[dbuf[p].at[pl.ds(0, 128)]],
                        ssem[p]).wait()

        issue_load(0, 0)

        def step(t, carry):
            for j3 in range(3):
                k = 3 * t + j3
                g = wid + 32 * k

                @pl.when((k >= 2) & (wid + 32 * (k - 2) < n_groups))
                def _():
                    drain_scatters((j3 + 1) % 3)

                issue_load(k + 1, (j3 + 1) % 3)

                @pl.when(g < n_groups)
                def _():
                    wait_load(j3)
                    if mode == "t":
                        for i in range(kr * 8):
                            idx = sbuf[j3][pl.ds(i * 16, 16)]
                            vbuf[0][j3][pl.ds(i * 16, 16)] = (
                                plsc.load_gather(tab, [idx]))
                    elif mode == "z":
                        for i in range(kr * 8):
                            idx = sbuf[j3][pl.ds(i * 16, 16)]
                            packed = plsc.load_gather(tab, [idx])
                            pair = plsc.bitcast(packed, jnp.bfloat16)
                            v0, v1 = plsc.unpack(
                                pair, format=plsc.PackFormat.INTERLEAVED)
                            vbuf[0][j3][pl.ds(i * 16, 16)] = v0
                            vbuf[1][j3][pl.ds(i * 16, 16)] = v1
                    for a in range(nacc):
                        for j in range(kr):
                            pltpu.async_copy(
                                vbuf[a][j3].at[pl.ds(j * 128, 128)],
                                acc[a].at[dbuf[j3].at[pl.ds(j * 128, 128)]],
                                ssem[j3], add=True)
            return carry

        lax.fori_loop(0, slots // 3, step, 0)
        plsc.subcore_barrier()

        @pl.when(s == 0)
        def _():
            for a in range(nacc):
                pltpu.sync_copy(acc[a], out_hbm.at[a, c])

    return pl.kernel(
        body,
        out_type=jax.ShapeDtypeStruct((nacc, 2, NP), jnp.float32),
        mesh=mesh,
        scratch_types=scratch,
        compiler_params=pltpu.CompilerParams(needs_layout_passes=False),
    )


_GRID = (NPR // 8,)
_BLK = pl.BlockSpec((8, 128), lambda g: (g, 0))
_SMEM = pl.BlockSpec(memory_space=pltpu.SMEM)


def _tc_a_body(s0, s1, x, dinv_o, y_o):
    deg = s0[...] + s1[...] + 1.0
    dinv = lax.rsqrt(deg)
    dinv_o[...] = dinv
    y_o[...] = x[...] * dinv


@jax.jit
def _tc_a(s0, s1, x):
    return pl.pallas_call(
        _tc_a_body,
        grid=_GRID,
        in_specs=[_BLK, _BLK, _BLK],
        out_specs=[_BLK, _BLK],
        out_shape=[jax.ShapeDtypeStruct((NPR, 128), jnp.float32)] * 2,
    )(s0, s1, x)


def _tc_b_body(s0, s1, dinv_r, x, w1, b1, w2, z0_o, z1_o, zp_o):
    dinv = dinv_r[...]
    t = dinv * (s0[...] + s1[...]) + x[...] * dinv * dinv
    a0 = jnp.zeros_like(t)
    a1 = jnp.zeros_like(t)
    for j in range(16):
        h = jnp.maximum(t * w1[0, j] + b1[j], 0.0)
        a0 = a0 + h * w2[j, 0]
        a1 = a1 + h * w2[j, 1]
    z0 = a0 * dinv
    z1 = a1 * dinv
    z0_o[...] = z0
    z1_o[...] = z1
    u0 = lax.bitcast_convert_type(
        z0.astype(jnp.bfloat16), jnp.uint16).astype(jnp.uint32)
    u1 = lax.bitcast_convert_type(
        z1.astype(jnp.bfloat16), jnp.uint16).astype(jnp.uint32)
    zp_o[...] = lax.bitcast_convert_type(u0 | (u1 << 16), jnp.int32)


@jax.jit
def _tc_b(s0, s1, dinv, x, W1, b1, W2):
    return pl.pallas_call(
        _tc_b_body,
        grid=_GRID,
        in_specs=[_BLK, _BLK, _BLK, _BLK, _SMEM, _SMEM, _SMEM],
        out_specs=[_BLK, _BLK, _BLK],
        out_shape=[jax.ShapeDtypeStruct((NPR, 128), jnp.float32)] * 2
        + [jax.ShapeDtypeStruct((NPR, 128), jnp.int32)],
    )(s0, s1, dinv, x, W1, b1, W2)


def _tc_c_body(p00, p01, p10, p11, dinv_r, z0, z1, b2, o0, o1):
    dinv = dinv_r[...]
    a0 = dinv * (p00[...] + p01[...]) + z0[...] * dinv + b2[0]
    a1 = dinv * (p10[...] + p11[...]) + z1[...] * dinv + b2[1]
    m = jnp.maximum(a0, a1)
    lse = m + jnp.log(jnp.exp(a0 - m) + jnp.exp(a1 - m))
    o0[...] = a0 - lse
    o1[...] = a1 - lse


@jax.jit
def _tc_c(p00, p01, p10, p11, dinv, z0, z1, b2):
    return pl.pallas_call(
        _tc_c_body,
        grid=_GRID,
        in_specs=[_BLK] * 7 + [_SMEM],
        out_specs=[_BLK, _BLK],
        out_shape=[jax.ShapeDtypeStruct((NPR, 128), jnp.float32)] * 2,
    )(p00, p01, p10, p11, dinv, z0, z1, b2)


def kernel(x, edge_index, W1, b1, W2, b2):
    ei = edge_index.astype(jnp.int32)
    zeros = jnp.zeros((NP,), jnp.float32)
    xp = jnp.pad(x[:, 0], (0, NP - N_NODES)).reshape(NPR, 128)

    degp = _sc_edge_pass("deg")(ei, zeros)
    dinv, y = _tc_a(degp[0, 0].reshape(NPR, 128),
                    degp[0, 1].reshape(NPR, 128), xp)

    sp = _sc_edge_pass("t")(y.reshape(NP), ei, zeros)
    z0, z1, zp = _tc_b(sp[0, 0].reshape(NPR, 128), sp[0, 1].reshape(NPR, 128),
                       dinv, xp, W1, b1, W2)

    p = _sc_edge_pass("z")(zp.reshape(NP), ei, zeros)
    o0, o1 = _tc_c(p[0, 0].reshape(NPR, 128), p[0, 1].reshape(NPR, 128),
                   p[1, 0].reshape(NPR, 128), p[1, 1].reshape(NPR, 128),
                   dinv, z0, z1, b2)
    return jnp.stack([o0.reshape(NP)[:N_NODES],
                      o1.reshape(NP)[:N_NODES]], axis=-1)


# speedup vs baseline: 366.2032x; 1.0179x over previous
import functools

import jax
import jax.numpy as jnp
from jax import lax
from jax.experimental import pallas as pl
from jax.experimental.pallas import tpu as pltpu
from jax.experimental.pallas import tpu_sc as plsc

N_NODES = 100000
N_EDGES = 6400000

LANES = 128
NP = 100352
NPR = NP // LANES

NW = 32
ROWS_TOTAL = N_EDGES // 128
CHUNKS = 8


def _nrsqrt(d):
    i = plsc.bitcast(d, jnp.int32)
    i = jnp.int32(0x5F3759DF) - lax.shift_right_arithmetic(i, 1)
    r = plsc.bitcast(i, jnp.float32)
    h = 0.5 * d
    for _ in range(3):
        r = r * (1.5 - h * r * r)
    return r


@functools.lru_cache(maxsize=None)
def _sc_edge_pass(mode):
    with_table = mode != "deg"
    nacc = 2 if mode == "z" else 1
    kr = 8 if mode == "z" else 16
    n_groups = ROWS_TOTAL // kr
    slots = -(-(-(-n_groups // 32)) // 3) * 3
    mesh = plsc.VectorSubcoreMesh(core_axis_name="c", subcore_axis_name="s")
    scratch = []
    if with_table:
        scratch.append(pltpu.VMEM((NP,), jnp.float32))
        scratch.append(pltpu.VMEM((1024,), jnp.float32))
        scratch += [pltpu.VMEM((kr * 128,), jnp.int32)] * 3
    scratch += [pltpu.VMEM((kr * 128,), jnp.int32)] * 3
    scratch += [pltpu.VMEM((kr * 128,), jnp.float32)] * (3 * nacc)
    scratch += [pltpu.VMEM_SHARED((NP,), jnp.float32)] * nacc
    scratch += [pltpu.SemaphoreType.DMA] * 3
    scratch += [pltpu.SemaphoreType.DMA] * 3

    def body(*refs):
        it = iter(refs)
        if mode == "t":
            deg_hbm = next(it)
            x_hbm = next(it)
        elif mode == "z":
            tp_hbm = next(it)
            deg_hbm = next(it)
            x_hbm = next(it)
            w_hbm = next(it)
        ei_hbm = next(it)
        zeros_hbm = next(it)
        out_hbm = next(it)
        if with_table:
            tab_hbm = next(it)
            tab = next(it)
            wsm = next(it)
            sbuf = (next(it), next(it), next(it))
        dbuf = (next(it), next(it), next(it))
        vbuf = tuple(tuple(next(it) for _ in range(3)) for _ in range(nacc))
        acc = tuple(next(it) for _ in range(nacc))
        lsem = (next(it), next(it), next(it))
        ssem = (next(it), next(it), next(it))
        c = lax.axis_index("c")
        s = lax.axis_index("s")
        wid = c * 16 + s

        @pl.when(s == 0)
        def _():
            for a in acc:
                pltpu.sync_copy(zeros_hbm, a)

        if with_table:
            if mode == "z":
                pltpu.sync_copy(w_hbm, wsm)
            csz = NP // 16 // CHUNKS
            nv = csz // 16

            def build_chunk(ch, carry):
                base = s * (NP // 16) + ch * csz
                if mode == "t":
                    b0, b1_, bx = vbuf[0][0], vbuf[0][1], vbuf[0][2]
                    pltpu.sync_copy(deg_hbm.at[pl.ds(base, csz)],
                                    b0.at[pl.ds(0, csz)])
                    pltpu.sync_copy(deg_hbm.at[pl.ds(NP + base, csz)],
                                    b1_.at[pl.ds(0, csz)])
                    pltpu.sync_copy(x_hbm.at[pl.ds(base, csz)],
                                    bx.at[pl.ds(0, csz)])

                    def it1(i, cc):
                        sl = pl.ds(i * 16, 16)
                        dinv = _nrsqrt(b0[sl] + b1_[sl] + 1.0)
                        bx[sl] = bx[sl] * dinv
                        return cc

                    lax.fori_loop(0, nv, it1, 0)
                    outb = bx
                else:
                    a0b, a1b, bdi = vbuf[0][0], vbuf[0][1], vbuf[0][2]
                    bout, btv = vbuf[1][0], vbuf[1][1]
                    pltpu.sync_copy(tp_hbm.at[pl.ds(base, csz)],
                                    a0b.at[pl.ds(0, csz)])
                    pltpu.sync_copy(tp_hbm.at[pl.ds(NP + base, csz)],
                                    a1b.at[pl.ds(0, csz)])
                    pltpu.sync_copy(deg_hbm.at[pl.ds(base, csz)],
                                    bdi.at[pl.ds(0, csz)])
                    pltpu.sync_copy(deg_hbm.at[pl.ds(NP + base, csz)],
                                    bout.at[pl.ds(0, csz)])
                    pltpu.sync_copy(x_hbm.at[pl.ds(base, csz)],
                                    btv.at[pl.ds(0, csz)])
                    zv = jnp.zeros((16,), jnp.float32)

                    def it1(i, cc):
                        sl = pl.ds(i * 16, 16)
                        dinv = _nrsqrt(bdi[sl] + bout[sl] + 1.0)
                        btv[sl] = dinv * (a0b[sl] + a1b[sl]) \
                            + btv[sl] * dinv * dinv
                        bdi[sl] = dinv
                        a0b[sl] = zv
                        a1b[sl] = zv
                        return cc

                    lax.fori_loop(0, nv, it1, 0)
                    for j in range(16):
                        w1vj = wsm[pl.ds(j * 16, 16)]
                        b1vj = wsm[pl.ds(256 + j * 16, 16)]
                        w20vj = wsm[pl.ds(512 + j * 16, 16)]
                        w21vj = wsm[pl.ds(768 + j * 16, 16)]

                        def itj(i, cc):
                            sl = pl.ds(i * 16, 16)
                            h = jnp.maximum(btv[sl] * w1vj + b1vj, 0.0)
                            a0b[sl] = a0b[sl] + h * w20vj
                            a1b[sl] = a1b[sl] + h * w21vj
                            return cc

                        lax.fori_loop(0, nv, itj, 0)

                    def it3(i, cc):
                        sl = pl.ds(i * 16, 16)
                        dinv = bdi[sl]
                        pair = plsc.pack(a0b[sl] * dinv, a1b[sl] * dinv,
                                         format=plsc.PackFormat.INTERLEAVED)
                        bout[sl] = plsc.bitcast(pair, jnp.float32)
                        return cc

                    lax.fori_loop(0, nv, it3, 0)
                    outb = bout
                pltpu.sync_copy(outb.at[pl.ds(0, csz)],
                                tab_hbm.at[pl.ds(base, csz)])
                return carry

            lax.fori_loop(0, CHUNKS, build_chunk, 0)
            plsc.subcore_barrier()
            pltpu.sync_copy(tab_hbm, tab)
        else:
            ones = jnp.full((16,), 1.0, jnp.float32)
            for p in range(3):
                for i in range(kr * 8):
                    vbuf[0][p][pl.ds(i * 16, 16)] = ones
        plsc.subcore_barrier()

        def issue_load(k, p):
            g = wid + 32 * k

            @pl.when(g < n_groups)
            def _():
                e0 = g * (kr * 128)
                pltpu.async_copy(
                    ei_hbm.at[1, pl.ds(e0, kr * 128)], dbuf[p], lsem[p])
                if with_table:
                    pltpu.async_copy(
                        ei_hbm.at[0, pl.ds(e0, kr * 128)], sbuf[p], lsem[p])

        def wait_load(p):
            pltpu.make_async_copy(
                ei_hbm.at[1, pl.ds(0, kr * 128)], dbuf[p], lsem[p]).wait()
            if with_table:
                pltpu.make_async_copy(
                    ei_hbm.at[0, pl.ds(0, kr * 128)], sbuf[p], lsem[p]).wait()

        def drain_scatters(p):
            for a in range(nacc):
                for j in range(kr):
                    pltpu.make_async_copy(
                        vbuf[a][p].at[pl.ds(0, 128)],
                        acc[a].at[dbuf[p].at[pl.ds(0, 128)]],
                        ssem[p]).wait()

        issue_load(0, 0)

        def step(t, carry):
            for j3 in range(3):
                k = 3 * t + j3
                g = wid + 32 * k

                @pl.when((k >= 2) & (wid + 32 * (k - 2) < n_groups))
                def _():
                    drain_scatters((j3 + 1) % 3)

                issue_load(k + 1, (j3 + 1) % 3)

                @pl.when(g < n_groups)
                def _():
                    wait_load(j3)
                    if mode == "t":
                        for i in range(kr * 8):
                            idx = sbuf[j3][pl.ds(i * 16, 16)]
                            vbuf[0][j3][pl.ds(i * 16, 16)] = (
                                plsc.load_gather(tab, [idx]))
                    elif mode == "z":
                        for i in range(kr * 8):
                            idx = sbuf[j3][pl.ds(i * 16, 16)]
                            packed = plsc.load_gather(tab, [idx])
                            pair = plsc.bitcast(packed, jnp.bfloat16)
                            v0, v1 = plsc.unpack(
                                pair, format=plsc.PackFormat.INTERLEAVED)
                            vbuf[0][j3][pl.ds(i * 16, 16)] = v0
                            vbuf[1][j3][pl.ds(i * 16, 16)] = v1
                    for a in range(nacc):
                        for j in range(kr):
                            pltpu.async_copy(
                                vbuf[a][j3].at[pl.ds(j * 128, 128)],
                                acc[a].at[dbuf[j3].at[pl.ds(j * 128, 128)]],
                                ssem[j3], add=True)
            return carry

        lax.fori_loop(0, slots // 3, step, 0)
        plsc.subcore_barrier()

        @pl.when(s == 0)
        def _():
            for a in range(nacc):
                pltpu.sync_copy(acc[a], out_hbm.at[a, c])

    out_type = jax.ShapeDtypeStruct((nacc, 2, NP), jnp.float32)
    if with_table:
        out_type = (out_type, jax.ShapeDtypeStruct((NP,), jnp.float32))
    return pl.kernel(
        body,
        out_type=out_type,
        mesh=mesh,
        scratch_types=scratch,
        compiler_params=pltpu.CompilerParams(needs_layout_passes=False),
    )


_GRID = (NPR // 8,)
_BLK = pl.BlockSpec((8, 128), lambda g: (g, 0))
_SMEM = pl.BlockSpec(memory_space=pltpu.SMEM)


def _tc_fin_body(d0, d1, t0, t1, p00, p01, p10, p11, x, w1, b1, w2, b2,
                 o0, o1):
    deg = d0[...] + d1[...] + 1.0
    dinv = lax.rsqrt(deg)
    tv = dinv * (t0[...] + t1[...]) + x[...] * dinv * dinv
    z0 = jnp.zeros_like(tv)
    z1 = jnp.zeros_like(tv)
    for j in range(16):
        h = jnp.maximum(tv * w1[0, j] + b1[j], 0.0)
        z0 = z0 + h * w2[j, 0]
        z1 = z1 + h * w2[j, 1]
    a0 = dinv * (p00[...] + p01[...]) + z0 * dinv * dinv + b2[0]
    a1 = dinv * (p10[...] + p11[...]) + z1 * dinv * dinv + b2[1]
    m = jnp.maximum(a0, a1)
    lse = m + jnp.log(jnp.exp(a0 - m) + jnp.exp(a1 - m))
    o0[...] = a0 - lse
    o1[...] = a1 - lse


@jax.jit
def _tc_fin(d0, d1, t0, t1, p00, p01, p10, p11, x, W1, b1, W2, b2):
    return pl.pallas_call(
        _tc_fin_body,
        grid=_GRID,
        in_specs=[_BLK] * 9 + [_SMEM] * 4,
        out_specs=[_BLK, _BLK],
        out_shape=[jax.ShapeDtypeStruct((NPR, 128), jnp.float32)] * 2,
    )(d0, d1, t0, t1, p00, p01, p10, p11, x, W1, b1, W2, b2)


def kernel(x, edge_index, W1, b1, W2, b2):
    ei = edge_index.astype(jnp.int32)
    zeros = jnp.zeros((NP,), jnp.float32)
    x1 = jnp.pad(x[:, 0], (0, NP - N_NODES))
    xp = x1.reshape(NPR, 128)
    wpk = jnp.broadcast_to(
        jnp.concatenate([W1[0], b1, W2[:, 0], W2[:, 1]])[:, None],
        (64, 16)).reshape(1024)

    degp = _sc_edge_pass("deg")(ei, zeros)
    degf = degp.reshape(2 * NP)
    tp, _yt = _sc_edge_pass("t")(degf, x1, ei, zeros)
    tpf = tp.reshape(2 * NP)
    zp, _zt = _sc_edge_pass("z")(tpf, degf, x1, wpk, ei, zeros)

    r = lambda a: a.reshape(NPR, 128)
    o0, o1 = _tc_fin(r(degp[0, 0]), r(degp[0, 1]), r(tp[0, 0]), r(tp[0, 1]),
                     r(zp[0, 0]), r(zp[0, 1]), r(zp[1, 0]), r(zp[1, 1]),
                     xp, W1, b1, W2, b2)
    return jnp.stack([o0.reshape(NP)[:N_NODES],
                      o1.reshape(NP)[:N_NODES]], axis=-1)


# speedup vs baseline: 407.8034x; 1.1136x over previous
import functools

import jax
import jax.numpy as jnp
from jax import lax
from jax.experimental import pallas as pl
from jax.experimental.pallas import tpu as pltpu
from jax.experimental.pallas import tpu_sc as plsc

N_NODES = 100000
N_EDGES = 6400000

LANES = 128
NP = 100352
NPR = NP // LANES

NW = 32
ROWS_TOTAL = N_EDGES // 128
CHUNKS = 8


def _nrsqrt(d):
    i = plsc.bitcast(d, jnp.int32)
    i = jnp.int32(0x5F3759DF) - lax.shift_right_arithmetic(i, 1)
    r = plsc.bitcast(i, jnp.float32)
    h = 0.5 * d
    for _ in range(3):
        r = r * (1.5 - h * r * r)
    return r


@functools.lru_cache(maxsize=None)
def _sc_edge_pass(mode):
    with_table = mode != "deg"
    nacc = 2 if mode == "z" else 1
    kr = 8 if mode == "z" else 16
    n_groups = ROWS_TOTAL // kr
    slots = -(-(-(-n_groups // 32)) // 3) * 3
    mesh = plsc.VectorSubcoreMesh(core_axis_name="c", subcore_axis_name="s")
    scratch = []
    if with_table:
        scratch.append(pltpu.VMEM((NP,), jnp.float32))
        scratch.append(pltpu.VMEM((1024,), jnp.float32))
        scratch += [pltpu.VMEM((kr * 128,), jnp.int32)] * 3
    scratch += [pltpu.VMEM((kr * 128,), jnp.int32)] * 3
    scratch += [pltpu.VMEM((kr * 128,), jnp.float32)] * (3 * nacc)
    scratch += [pltpu.VMEM_SHARED((NP,), jnp.float32)] * nacc
    scratch += [pltpu.SemaphoreType.DMA] * 3
    scratch += [pltpu.SemaphoreType.DMA] * 3

    def body(*refs):
        it = iter(refs)
        if mode == "t":
            deg_hbm = next(it)
            x_hbm = next(it)
        elif mode == "z":
            tp_hbm = next(it)
            deg_hbm = next(it)
            x_hbm = next(it)
            w_hbm = next(it)
        ei_hbm = next(it)
        zeros_hbm = next(it)
        out_hbm = next(it)
        if with_table:
            tab_hbm = next(it)
            tab = next(it)
            wsm = next(it)
            sbuf = (next(it), next(it), next(it))
        dbuf = (next(it), next(it), next(it))
        vbuf = tuple(tuple(next(it) for _ in range(3)) for _ in range(nacc))
        acc = tuple(next(it) for _ in range(nacc))
        lsem = (next(it), next(it), next(it))
        ssem = (next(it), next(it), next(it))
        c = lax.axis_index("c")
        s = lax.axis_index("s")
        wid = c * 16 + s

        @pl.when(s == 0)
        def _():
            for a in acc:
                pltpu.sync_copy(zeros_hbm, a)

        if with_table:
            if mode == "z":
                pltpu.sync_copy(w_hbm, wsm)
            csz = NP // 16 // CHUNKS
            nv = csz // 16

            def build_chunk(ch, carry):
                base = s * (NP // 16) + ch * csz
                if mode == "t":
                    b0, b1_, bx = vbuf[0][0], vbuf[0][1], vbuf[0][2]
                    cps = [
                        pltpu.async_copy(deg_hbm.at[pl.ds(base, csz)],
                                         b0.at[pl.ds(0, csz)], lsem[0]),
                        pltpu.async_copy(deg_hbm.at[pl.ds(NP + base, csz)],
                                         b1_.at[pl.ds(0, csz)], lsem[0]),
                        pltpu.async_copy(x_hbm.at[pl.ds(base, csz)],
                                         bx.at[pl.ds(0, csz)], lsem[0]),
                    ]
                    for cp in cps:
                        cp.wait()

                    def it1(i, cc):
                        sl = pl.ds(i * 16, 16)
                        dinv = _nrsqrt(b0[sl] + b1_[sl] + 1.0)
                        bx[sl] = bx[sl] * dinv
                        return cc

                    lax.fori_loop(0, nv, it1, 0)
                    outb = bx
                else:
                    bt0, bt1, bd0, bd1, bx = (vbuf[0][0], vbuf[0][1],
                                              vbuf[0][2], vbuf[1][0],
                                              vbuf[1][1])
                    cps = [
                        pltpu.async_copy(tp_hbm.at[pl.ds(base, csz)],
                                         bt0.at[pl.ds(0, csz)], lsem[0]),
                        pltpu.async_copy(tp_hbm.at[pl.ds(NP + base, csz)],
                                         bt1.at[pl.ds(0, csz)], lsem[0]),
                        pltpu.async_copy(deg_hbm.at[pl.ds(base, csz)],
                                         bd0.at[pl.ds(0, csz)], lsem[0]),
                        pltpu.async_copy(deg_hbm.at[pl.ds(NP + base, csz)],
                                         bd1.at[pl.ds(0, csz)], lsem[0]),
                        pltpu.async_copy(x_hbm.at[pl.ds(base, csz)],
                                         bx.at[pl.ds(0, csz)], lsem[0]),
                    ]
                    for cp in cps:
                        cp.wait()

                    def it1(i, cc):
                        sl = pl.ds(i * 16, 16)
                        dinv = _nrsqrt(bd0[sl] + bd1[sl] + 1.0)
                        tv = dinv * (bt0[sl] + bt1[sl]) \
                            + bx[sl] * dinv * dinv
                        a0 = jnp.zeros((16,), jnp.float32)
                        a1 = jnp.zeros((16,), jnp.float32)
                        for j in range(16):
                            h = jnp.maximum(
                                tv * wsm[pl.ds(j * 16, 16)]
                                + wsm[pl.ds(256 + j * 16, 16)], 0.0)
                            a0 = a0 + h * wsm[pl.ds(512 + j * 16, 16)]
                            a1 = a1 + h * wsm[pl.ds(768 + j * 16, 16)]
                        pair = plsc.pack(a0 * dinv, a1 * dinv,
                                         format=plsc.PackFormat.INTERLEAVED)
                        bx[sl] = plsc.bitcast(pair, jnp.float32)
                        return cc

                    lax.fori_loop(0, nv, it1, 0)
                    outb = bx
                pltpu.sync_copy(outb.at[pl.ds(0, csz)],
                                tab_hbm.at[pl.ds(base, csz)])
                return carry

            lax.fori_loop(0, CHUNKS, build_chunk, 0)
            plsc.subcore_barrier()
            pltpu.sync_copy(tab_hbm, tab)
        else:
            ones = jnp.full((16,), 1.0, jnp.float32)
            for p in range(3):
                for i in range(kr * 8):
                    vbuf[0][p][pl.ds(i * 16, 16)] = ones
        plsc.subcore_barrier()

        def issue_load(k, p):
            g = wid + 32 * k

            @pl.when(g < n_groups)
            def _():
                e0 = g * (kr * 128)
                pltpu.async_copy(
                    ei_hbm.at[1, pl.ds(e0, kr * 128)], dbuf[p], lsem[p])
                if with_table:
                    pltpu.async_copy(
                        ei_hbm.at[0, pl.ds(e0, kr * 128)], sbuf[p], lsem[p])

        def wait_load(p):
            pltpu.make_async_copy(
                ei_hbm.at[1, pl.ds(0, kr * 128)], dbuf[p], lsem[p]).wait()
            if with_table:
                pltpu.make_async_copy(
                    ei_hbm.at[0, pl.ds(0, kr * 128)], sbuf[p], lsem[p]).wait()

        def drain_scatters(p):
            for a in range(nacc):
                for j in range(kr):
                    pltpu.make_async_copy(
                        vbuf[a][p].at[pl.ds(0, 128)],
                        acc[a].at[dbuf[p].at[pl.ds(0, 128)]],
                        ssem[p]).wait()

        issue_load(0, 0)

        def step(t, carry):
            for j3 in range(3):
                k = 3 * t + j3
                g = wid + 32 * k

                @pl.when((k >= 2) & (wid + 32 * (k - 2) < n_groups))
                def _():
                    drain_scatters((j3 + 1) % 3)

                issue_load(k + 1, (j3 + 1) % 3)

                @pl.when(g < n_groups)
                def _():
                    wait_load(j3)
                    if mode == "t":
                        for i in range(kr * 8):
                            idx = sbuf[j3][pl.ds(i * 16, 16)]
                            vbuf[0][j3][pl.ds(i * 16, 16)] = (
                                plsc.load_gather(tab, [idx]))
                    elif mode == "z":
                        for i in range(kr * 8):
                            idx = sbuf[j3][pl.ds(i * 16, 16)]
                            packed = plsc.load_gather(tab, [idx])
                            pair = plsc.bitcast(packed, jnp.bfloat16)
                            v0, v1 = plsc.unpack(
                                pair, format=plsc.PackFormat.INTERLEAVED)
                            vbuf[0][j3][pl.ds(i * 16, 16)] = v0
                            vbuf[1][j3][pl.ds(i * 16, 16)] = v1
                    for a in range(nacc):
                        for j in range(kr):
                            pltpu.async_copy(
                                vbuf[a][j3].at[pl.ds(j * 128, 128)],
                                acc[a].at[dbuf[j3].at[pl.ds(j * 128, 128)]],
                                ssem[j3], add=True)
            return carry

        lax.fori_loop(0, slots // 3, step, 0)
        plsc.subcore_barrier()

        @pl.when(s == 0)
        def _():
            for a in range(nacc):
                pltpu.sync_copy(acc[a], out_hbm.at[a, c])

    out_type = jax.ShapeDtypeStruct((nacc, 2, NP), jnp.float32)
    if with_table:
        out_type = (out_type, jax.ShapeDtypeStruct((NP,), jnp.float32))
    return pl.kernel(
        body,
        out_type=out_type,
        mesh=mesh,
        scratch_types=scratch,
        compiler_params=pltpu.CompilerParams(needs_layout_passes=False),
    )


_GRID = (NPR // 8,)
_BLK = pl.BlockSpec((8, 128), lambda g: (g, 0))
_SMEM = pl.BlockSpec(memory_space=pltpu.SMEM)


def _tc_fin_body(d0, d1, t0, t1, p00, p01, p10, p11, x, w1, b1, w2, b2,
                 o0, o1):
    deg = d0[...] + d1[...] + 1.0
    dinv = lax.rsqrt(deg)
    tv = dinv * (t0[...] + t1[...]) + x[...] * dinv * dinv
    z0 = jnp.zeros_like(tv)
    z1 = jnp.zeros_like(tv)
    for j in range(16):
        h = jnp.maximum(tv * w1[0, j] + b1[j], 0.0)
        z0 = z0 + h * w2[j, 0]
        z1 = z1 + h * w2[j, 1]
    a0 = dinv * (p00[...] + p01[...]) + z0 * dinv * dinv + b2[0]
    a1 = dinv * (p10[...] + p11[...]) + z1 * dinv * dinv + b2[1]
    m = jnp.maximum(a0, a1)
    lse = m + jnp.log(jnp.exp(a0 - m) + jnp.exp(a1 - m))
    o0[...] = a0 - lse
    o1[...] = a1 - lse


@jax.jit
def _tc_fin(d0, d1, t0, t1, p00, p01, p10, p11, x, W1, b1, W2, b2):
    return pl.pallas_call(
        _tc_fin_body,
        grid=_GRID,
        in_specs=[_BLK] * 9 + [_SMEM] * 4,
        out_specs=[_BLK, _BLK],
        out_shape=[jax.ShapeDtypeStruct((NPR, 128), jnp.float32)] * 2,
    )(d0, d1, t0, t1, p00, p01, p10, p11, x, W1, b1, W2, b2)


def kernel(x, edge_index, W1, b1, W2, b2):
    ei = edge_index.astype(jnp.int32)
    zeros = jnp.zeros((NP,), jnp.float32)
    x1 = jnp.pad(x[:, 0], (0, NP - N_NODES))
    xp = x1.reshape(NPR, 128)
    wpk = jnp.broadcast_to(
        jnp.concatenate([W1[0], b1, W2[:, 0], W2[:, 1]])[:, None],
        (64, 16)).reshape(1024)

    degp = _sc_edge_pass("deg")(ei, zeros)
    degf = degp.reshape(2 * NP)
    tp, _yt = _sc_edge_pass("t")(degf, x1, ei, zeros)
    tpf = tp.reshape(2 * NP)
    zp, _zt = _sc_edge_pass("z")(tpf, degf, x1, wpk, ei, zeros)

    r = lambda a: a.reshape(NPR, 128)
    o0, o1 = _tc_fin(r(degp[0, 0]), r(degp[0, 1]), r(tp[0, 0]), r(tp[0, 1]),
                     r(zp[0, 0]), r(zp[0, 1]), r(zp[1, 0]), r(zp[1, 1]),
                     xp, W1, b1, W2, b2)
    return jnp.stack([o0.reshape(NP)[:N_NODES],
                      o1.reshape(NP)[:N_NODES]], axis=-1)
